# baseline (speedup 1.0000x reference)
import functools

import jax
import jax.numpy as jnp
from jax import lax
from jax.experimental import pallas as pl
from jax.experimental.pallas import tpu as pltpu

N_NODES = 10000
IN_DIM = 128
BM = 2000


def _mask_body(tok_ref, noi_ref, nsrc_ref, x_ref, mtok_ref, out_ref):
    out_ref[...] = x_ref[...]
    n_tok = tok_ref.shape[0]
    n_noi = noi_ref.shape[0]

    def tok_iter(i, carry):
        r = tok_ref[i]
        out_ref[pl.ds(r, 1), :] = mtok_ref[...]
        return carry

    lax.fori_loop(0, n_tok, tok_iter, 0, unroll=False)

    def noi_iter(i, carry):
        r = noi_ref[i]
        s = nsrc_ref[i]
        out_ref[pl.ds(r, 1), :] = x_ref[pl.ds(s, 1), :]
        return carry

    lax.fori_loop(0, n_noi, noi_iter, 0, unroll=False)


def _mask_nodes_tc(x, token_nodes, noise_nodes, noise_src, mask_token):
    return pl.pallas_call(
        _mask_body,
        out_shape=jax.ShapeDtypeStruct(x.shape, x.dtype),
        in_specs=[
            pl.BlockSpec(memory_space=pltpu.SMEM),
            pl.BlockSpec(memory_space=pltpu.SMEM),
            pl.BlockSpec(memory_space=pltpu.SMEM),
            pl.BlockSpec(memory_space=pltpu.VMEM),
            pl.BlockSpec(memory_space=pltpu.VMEM),
        ],
        out_specs=pl.BlockSpec(memory_space=pltpu.VMEM),
    )(token_nodes, noise_nodes, noise_src, x, mask_token)


def _mm_body(h_ref, w_ref, as_ref, ad_ref, hw_ref, es_ref, ed_ref, *, act):
    h = h_ref[...]
    if act == "elu":
        h = jnp.where(h > 0, h, jnp.exp(jnp.minimum(h, 0.0)) - 1.0)
    hw = jnp.dot(h, w_ref[...], preferred_element_type=jnp.float32)
    hw_ref[...] = hw
    es_ref[...] = jnp.sum(hw * as_ref[...], axis=1, keepdims=True)
    ed_ref[...] = jnp.sum(hw * ad_ref[...], axis=1, keepdims=True)


def _mm_es_ed(h, w, a_src, a_dst, act):
    n, din = h.shape
    dout = w.shape[1]
    grid = n // BM
    body = functools.partial(_mm_body, act=act)
    hw, es, ed = pl.pallas_call(
        body,
        grid=(grid,),
        in_specs=[
            pl.BlockSpec((BM, din), lambda i: (i, 0)),
            pl.BlockSpec((din, dout), lambda i: (0, 0)),
            pl.BlockSpec((1, dout), lambda i: (0, 0)),
            pl.BlockSpec((1, dout), lambda i: (0, 0)),
        ],
        out_specs=[
            pl.BlockSpec((BM, dout), lambda i: (i, 0)),
            pl.BlockSpec((BM, 1), lambda i: (i, 0)),
            pl.BlockSpec((BM, 1), lambda i: (i, 0)),
        ],
        out_shape=[
            jax.ShapeDtypeStruct((n, dout), jnp.float32),
            jax.ShapeDtypeStruct((n, 1), jnp.float32),
            jax.ShapeDtypeStruct((n, 1), jnp.float32),
        ],
    )(h, w, a_src.reshape(1, dout), a_dst.reshape(1, dout))
    return hw, es.reshape(n), ed.reshape(n)


def _edge_phase(hw, es, ed, src, dst, b, n):
    e = jax.nn.leaky_relu(es[src] + ed[dst], 0.2)
    emax = jax.ops.segment_max(e, dst, num_segments=n)
    emax = jnp.where(jnp.isfinite(emax), emax, 0.0)
    ex = jnp.exp(e - emax[dst])
    denom = jax.ops.segment_sum(ex, dst, num_segments=n)
    alpha = ex / (denom[dst] + 1e-16)
    out = jax.ops.segment_sum(hw[src] * alpha[:, None], dst, num_segments=n)
    return out + b


def _zero_rows_body(idx_ref, x_ref, out_ref):
    out_ref[...] = x_ref[...]
    zero = jnp.zeros((1, out_ref.shape[1]), jnp.float32)

    def it(i, carry):
        out_ref[pl.ds(idx_ref[i], 1), :] = zero
        return carry

    lax.fori_loop(0, idx_ref.shape[0], it, 0, unroll=False)


def _zero_rows_tc(x, idx):
    return pl.pallas_call(
        _zero_rows_body,
        out_shape=jax.ShapeDtypeStruct(x.shape, x.dtype),
        in_specs=[
            pl.BlockSpec(memory_space=pltpu.SMEM),
            pl.BlockSpec(memory_space=pltpu.VMEM),
        ],
        out_specs=pl.BlockSpec(memory_space=pltpu.VMEM),
    )(idx, x)


def _loss_body(m_ref, x_ref, r_ref, out_ref):
    k = m_ref.shape[0]

    def it(i, acc):
        idx = m_ref[i]
        xr = x_ref[pl.ds(idx, 1), :]
        rr = r_ref[pl.ds(idx, 1), :]
        nx = jnp.sqrt(jnp.sum(xr * xr)) + 1e-12
        nr = jnp.sqrt(jnp.sum(rr * rr)) + 1e-12
        c = jnp.sum(xr * rr) / (nx * nr)
        t = 1.0 - c
        return acc + t * t * t

    acc = lax.fori_loop(0, k, it, jnp.float32(0.0), unroll=False)
    out_ref[0] = acc / k


def _sce_loss_tc(x, recon, mask_nodes):
    out = pl.pallas_call(
        _loss_body,
        out_shape=jax.ShapeDtypeStruct((1,), jnp.float32),
        in_specs=[
            pl.BlockSpec(memory_space=pltpu.SMEM),
            pl.BlockSpec(memory_space=pltpu.VMEM),
            pl.BlockSpec(memory_space=pltpu.VMEM),
        ],
        out_specs=pl.BlockSpec(memory_space=pltpu.SMEM),
    )(mask_nodes, x, recon)
    return out[0]


def kernel(x, edge_index, mask_nodes, token_nodes, noise_nodes, noise_src, params):
    n = x.shape[0]
    src, dst = edge_index[0], edge_index[1]

    out_x = _mask_nodes_tc(x, token_nodes, noise_nodes, noise_src,
                           params["enc_mask_token"])

    hw0, es0, ed0 = _mm_es_ed(out_x, params["enc_W0"], params["enc_asrc0"],
                              params["enc_adst0"], act=None)
    out0 = _edge_phase(hw0, es0, ed0, src, dst, params["enc_b0"], n)

    w1p = jnp.zeros((256, 128), jnp.float32).at[:, :30].set(params["enc_W1"])
    as1p = jnp.zeros((128,), jnp.float32).at[:30].set(params["enc_asrc1"])
    ad1p = jnp.zeros((128,), jnp.float32).at[:30].set(params["enc_adst1"])
    b1p = jnp.zeros((128,), jnp.float32).at[:30].set(params["enc_b1"])
    hw1, es1, ed1 = _mm_es_ed(out0, w1p, as1p, ad1p, act="elu")
    out1 = _edge_phase(hw1, es1, ed1, src, dst, b1p, n)

    enc_rep = _zero_rows_tc(out1, mask_nodes)

    w2p = jnp.zeros((128, 256), jnp.float32).at[:30, :].set(params["dec_W0"])
    hw2, es2, ed2 = _mm_es_ed(enc_rep, w2p, params["dec_asrc0"],
                              params["dec_adst0"], act=None)
    out2 = _edge_phase(hw2, es2, ed2, src, dst, params["dec_b0"], n)

    hw3, es3, ed3 = _mm_es_ed(out2, params["dec_W1"], params["dec_asrc1"],
                              params["dec_adst1"], act="elu")
    recon = _edge_phase(hw3, es3, ed3, src, dst, params["dec_b1"], n)

    loss = _sce_loss_tc(x, recon, mask_nodes)
    return loss, recon


# speedup vs baseline: 5.8631x; 5.8631x over previous
import functools

import numpy as np

import jax
import jax.numpy as jnp
from jax import lax
from jax.experimental import pallas as pl
from jax.experimental.pallas import tpu as pltpu

N_NODES = 10000
IN_DIM = 128
BM = 2000


def _mask_body(tok_ref, noi_ref, nsrc_ref, x_ref, mtok_ref, out_ref):
    out_ref[...] = x_ref[...]
    n_tok = tok_ref.shape[0]
    n_noi = noi_ref.shape[0]

    def tok_iter(i, carry):
        r = tok_ref[i]
        out_ref[pl.ds(r, 1), :] = mtok_ref[...]
        return carry

    lax.fori_loop(0, n_tok, tok_iter, 0, unroll=False)

    def noi_iter(i, carry):
        r = noi_ref[i]
        s = nsrc_ref[i]
        out_ref[pl.ds(r, 1), :] = x_ref[pl.ds(s, 1), :]
        return carry

    lax.fori_loop(0, n_noi, noi_iter, 0, unroll=False)


def _mask_nodes_tc(x, token_nodes, noise_nodes, noise_src, mask_token):
    return pl.pallas_call(
        _mask_body,
        out_shape=jax.ShapeDtypeStruct(x.shape, x.dtype),
        in_specs=[
            pl.BlockSpec(memory_space=pltpu.SMEM),
            pl.BlockSpec(memory_space=pltpu.SMEM),
            pl.BlockSpec(memory_space=pltpu.SMEM),
            pl.BlockSpec(memory_space=pltpu.VMEM),
            pl.BlockSpec(memory_space=pltpu.VMEM),
        ],
        out_specs=pl.BlockSpec(memory_space=pltpu.VMEM),
    )(token_nodes, noise_nodes, noise_src, x, mask_token)


def _mm_body(h_ref, w_ref, as_ref, ad_ref, hw_ref, es_ref, ed_ref, *, act):
    h = h_ref[...]
    if act == "elu":
        h = jnp.where(h > 0, h, jnp.exp(jnp.minimum(h, 0.0)) - 1.0)
    hw = jnp.dot(h, w_ref[...], preferred_element_type=jnp.float32)
    hw_ref[...] = hw
    es_ref[...] = jnp.sum(hw * as_ref[...], axis=1, keepdims=True)
    ed_ref[...] = jnp.sum(hw * ad_ref[...], axis=1, keepdims=True)


def _mm_es_ed(h, w, a_src, a_dst, act):
    n, din = h.shape
    dout = w.shape[1]
    grid = n // BM
    body = functools.partial(_mm_body, act=act)
    hw, es, ed = pl.pallas_call(
        body,
        grid=(grid,),
        in_specs=[
            pl.BlockSpec((BM, din), lambda i: (i, 0)),
            pl.BlockSpec((din, dout), lambda i: (0, 0)),
            pl.BlockSpec((1, dout), lambda i: (0, 0)),
            pl.BlockSpec((1, dout), lambda i: (0, 0)),
        ],
        out_specs=[
            pl.BlockSpec((BM, dout), lambda i: (i, 0)),
            pl.BlockSpec((BM, 1), lambda i: (i, 0)),
            pl.BlockSpec((BM, 1), lambda i: (i, 0)),
        ],
        out_shape=[
            jax.ShapeDtypeStruct((n, dout), jnp.float32),
            jax.ShapeDtypeStruct((n, 1), jnp.float32),
            jax.ShapeDtypeStruct((n, 1), jnp.float32),
        ],
    )(h, w, a_src.reshape(1, dout), a_dst.reshape(1, dout))
    return hw, es.reshape(n), ed.reshape(n)


NTILES = 32
NC = 2
NPT = 313
N_PAD = NTILES * NPT
CH = 2048
_NEG = np.float32(-3e38)


def _vgather(v, idx):
    return v.at[idx].get(mode="promise_in_bounds")


def _seg_scan(key, val, op):
    lane = lax.iota(jnp.int32, 16)
    for s in (1, 2, 4, 8):
        idx = jnp.maximum(lane - s, 0)
        pv = _vgather(val, idx)
        pk = _vgather(key, idx)
        take = (lane >= s) & (pk == key)
        val = jnp.where(take, op(val, pv), val)
    nxt = _vgather(key, jnp.minimum(lane + 1, 15))
    is_last = (lane == 15) | (nxt != key)
    return val, is_last


def _make_edge_kernel(d):
    from jax.experimental.pallas import tpu_sc as plsc

    mesh = plsc.VectorSubcoreMesh(core_axis_name="c", subcore_axis_name="s")
    ncb = d // 16

    def body(hw, es, ed, srcp, dstp, starts, bias, out,
             es_v, ed_v, st_v, bias_v, cur_v, acc_v, tmp_v,
             src_st, dst_st, rows0_v, rows1_v, sem0, sem1):
        wid = lax.axis_index("s") * NC + lax.axis_index("c")
        base = wid * NPT

        pltpu.sync_copy(starts, st_v.at[pl.ds(0, 40)])
        zero16 = jnp.zeros((16,), jnp.float32)
        zero16i = jnp.zeros((16,), jnp.int32)
        negv = jnp.full((16,), _NEG)
        stc = [st_v[pl.ds(c * 16, 16)] for c in range(3)]
        s_lo = stc[0][0] * 0
        s_hi = s_lo
        for j in range(33):
            c, l = divmod(j, 16)
            s_lo = jnp.where(wid == j, stc[c][l], s_lo)
            s_hi = jnp.where(wid + 1 == j, stc[c][l], s_hi)
        a0 = (s_lo // 16) * 16
        nch = (s_hi - a0 + (CH - 1)) // CH

        pltpu.sync_copy(es, es_v.at[pl.ds(0, N_NODES)])
        pltpu.sync_copy(ed, ed_v.at[pl.ds(0, N_NODES)])
        es_v[pl.ds(N_NODES, 16)] = zero16
        ed_v[pl.ds(N_NODES, 16)] = zero16
        pltpu.sync_copy(bias, bias_v)
        bias_chunks = [bias_v[pl.ds(c * 16, 16)] for c in range(ncb)]
        for c in range(ncb):
            cur_v[pl.ds(c * 16, 16)] = zero16
        for rr in range(16):
            for c in range(ncb):
                rows0_v[rr, pl.ds(c * 16, 16)] = zero16
                rows1_v[rr, pl.ds(c * 16, 16)] = zero16

        def init_row(r, carry):
            ro = pl.multiple_of(r * d, 16)
            for c in range(ncb):
                acc_v[pl.ds(ro + c * 16, 16)] = bias_chunks[c]
            return carry

        lax.fori_loop(0, NPT, init_row, 0, unroll=False)

        def lookup(tab, i):
            b8 = pl.multiple_of((i // 16) * 16, 16)
            w = tab[pl.ds(b8, 16)]
            return _vgather(w, jnp.full((16,), i - b8, jnp.int32))

        def finalize(prev_dl, s_run):
            iv = 1.0 / (s_run + np.float32(1e-16))
            ro = pl.multiple_of(prev_dl * d, 16)
            for c in range(ncb):
                acc_v[pl.ds(ro + c * 16, 16)] = (
                    bias_chunks[c] + cur_v[pl.ds(c * 16, 16)] * iv)

        def fire(g, rows, sem_):
            lo = pl.multiple_of(g * 16, 16)
            pltpu.async_copy(hw.at[src_st.at[pl.ds(lo, 16)]], rows, sem_)

        def wait(g, rows, sem_):
            lo = pl.multiple_of(g * 16, 16)
            pltpu.make_async_copy(hw.at[src_st.at[pl.ds(lo, 16)]],
                                  rows, sem_).wait()

        def chunk_loop(c, carry):
            off = a0 + c * CH
            pltpu.sync_copy(srcp.at[pl.ds(off, CH)], src_st)
            pltpu.sync_copy(dstp.at[pl.ds(off, CH)], dst_st)
            rem = s_hi - off
            ngr = jnp.clip((rem + 15) // 16, 0, CH // 16)
            npair = (ngr + 1) // 2

            def process(g, rows_ref, carry2):
                lo = pl.multiple_of(g * 16, 16)
                srcv = src_st[pl.ds(lo, 16)]
                dstv = dst_st[pl.ds(lo, 16)]
                gid = off + lo + lax.iota(jnp.int32, 16)
                validv = jnp.where((gid >= s_lo) & (gid < s_hi),
                                   jnp.int32(1), jnp.int32(0))
                dlv = dstv - base

                m_v, s_v, prev = carry2
                for t in range(16):
                    k_t = dlv[t]
                    v_t = validv[t] != 0
                    es_b = lookup(es_v, srcv[t])
                    ed_b = lookup(ed_v, dstv[t])
                    e_b = es_b + ed_b
                    e_b = jnp.where(e_b > 0, e_b, e_b * np.float32(0.2))
                    is_new = v_t & (k_t != prev)

                    @pl.when(is_new & (prev >= 0))
                    def _():
                        finalize(prev, s_v)

                    m0 = jnp.where(is_new, negv, m_v)
                    s0 = jnp.where(is_new, zero16, s_v)
                    m1 = jnp.where(v_t, jnp.maximum(m0, e_b), m0)
                    rv = jnp.exp(m0 - m1)
                    wv = jnp.where(v_t, jnp.exp(e_b - m1), zero16)

                    for cc in range(ncb):
                        sl = pl.ds(cc * 16, 16)
                        cur_v[sl] = cur_v[sl] * rv + wv * rows_ref[t, sl]

                    m_v = m1
                    s_v = s0 * rv + wv
                    prev = jnp.where(v_t, k_t, prev)
                return m_v, s_v, prev

            @pl.when(npair > 0)
            def _():
                fire(0, rows0_v, sem0)

            def pair_loop(j, carry2):
                g0 = 2 * j

                @pl.when(g0 + 1 < ngr)
                def _():
                    fire(g0 + 1, rows1_v, sem1)

                wait(g0, rows0_v, sem0)
                carry3 = process(g0, rows0_v, carry2)

                @pl.when(g0 + 2 < ngr)
                def _():
                    fire(g0 + 2, rows0_v, sem0)

                @pl.when(g0 + 1 < ngr)
                def _():
                    wait(g0 + 1, rows1_v, sem1)
                return process(g0 + 1, rows1_v, carry3)

            return lax.fori_loop(0, npair, pair_loop, carry,
                                 unroll=False)

        m_v, s_v, prev = lax.fori_loop(
            0, nch, chunk_loop,
            (negv, zero16, jnp.int32(-1)),
            unroll=False)

        @pl.when(prev >= 0)
        def _():
            finalize(prev, s_v)

        pltpu.sync_copy(acc_v, out.at[pl.ds(base * d, NPT * d)])

    return pl.kernel(
        body,
        out_type=jax.ShapeDtypeStruct((N_PAD * d,), jnp.float32),
        mesh=mesh,
        scratch_types=[
            pltpu.VMEM((N_PAD,), jnp.float32),
            pltpu.VMEM((N_PAD,), jnp.float32),
            pltpu.VMEM((48,), jnp.int32),
            pltpu.VMEM((d,), jnp.float32),
            pltpu.VMEM((d,), jnp.float32),
            pltpu.VMEM((NPT * d,), jnp.float32),
            pltpu.VMEM((16,), jnp.float32),
            pltpu.VMEM((CH,), jnp.int32),
            pltpu.VMEM((CH,), jnp.int32),
            pltpu.VMEM((16, d), jnp.float32),
            pltpu.VMEM((16, d), jnp.float32),
            pltpu.SemaphoreType.DMA,
            pltpu.SemaphoreType.DMA,
        ],
    )


_EDGE_KERNELS = {}


def _edge_phase_sc(hw, es, ed, srcp, dstp, starts, b):
    d = hw.shape[1]
    if d not in _EDGE_KERNELS:
        _EDGE_KERNELS[d] = _make_edge_kernel(d)
    out = _EDGE_KERNELS[d](hw, es, ed, srcp, dstp, starts, b)
    return out.reshape(N_PAD, d)[:N_NODES]


def _zero_rows_body(idx_ref, x_ref, out_ref):
    out_ref[...] = x_ref[...]
    zero = jnp.zeros((1, out_ref.shape[1]), jnp.float32)

    def it(i, carry):
        out_ref[pl.ds(idx_ref[i], 1), :] = zero
        return carry

    lax.fori_loop(0, idx_ref.shape[0], it, 0, unroll=False)


def _zero_rows_tc(x, idx):
    return pl.pallas_call(
        _zero_rows_body,
        out_shape=jax.ShapeDtypeStruct(x.shape, x.dtype),
        in_specs=[
            pl.BlockSpec(memory_space=pltpu.SMEM),
            pl.BlockSpec(memory_space=pltpu.VMEM),
        ],
        out_specs=pl.BlockSpec(memory_space=pltpu.VMEM),
    )(idx, x)


def _loss_body(m_ref, x_ref, r_ref, out_ref):
    k = m_ref.shape[0]

    def it(i, acc):
        idx = m_ref[i]
        xr = x_ref[pl.ds(idx, 1), :]
        rr = r_ref[pl.ds(idx, 1), :]
        nx = jnp.sqrt(jnp.sum(xr * xr)) + 1e-12
        nr = jnp.sqrt(jnp.sum(rr * rr)) + 1e-12
        c = jnp.sum(xr * rr) / (nx * nr)
        t = 1.0 - c
        return acc + t * t * t

    acc = lax.fori_loop(0, k, it, np.float32(0.0), unroll=False)
    out_ref[0] = acc / k


def _sce_loss_tc(x, recon, mask_nodes):
    out = pl.pallas_call(
        _loss_body,
        out_shape=jax.ShapeDtypeStruct((1,), jnp.float32),
        in_specs=[
            pl.BlockSpec(memory_space=pltpu.SMEM),
            pl.BlockSpec(memory_space=pltpu.VMEM),
            pl.BlockSpec(memory_space=pltpu.VMEM),
        ],
        out_specs=pl.BlockSpec(memory_space=pltpu.SMEM),
    )(mask_nodes, x, recon)
    return out[0]


def kernel(x, edge_index, mask_nodes, token_nodes, noise_nodes, noise_src, params):
    n = x.shape[0]
    src, dst = edge_index[0], edge_index[1]

    sdst, ssrc = lax.sort((dst, src), num_keys=1)
    starts = jnp.searchsorted(
        sdst, np.arange(0, N_PAD + 1, NPT)).astype(jnp.int32)
    starts = jnp.pad(starts, (0, 40 - starts.shape[0]))
    sdst_p = jnp.concatenate(
        [sdst, np.full((CH,), N_PAD - 1, np.int32)])
    ssrc_p = jnp.concatenate([ssrc, np.zeros((CH,), np.int32)])

    out_x = _mask_nodes_tc(x, token_nodes, noise_nodes, noise_src,
                           params["enc_mask_token"])

    hw0, es0, ed0 = _mm_es_ed(out_x, params["enc_W0"], params["enc_asrc0"],
                              params["enc_adst0"], act=None)
    out0 = _edge_phase_sc(hw0, es0, ed0, ssrc_p, sdst_p, starts, params["enc_b0"])

    w1p = jnp.pad(params["enc_W1"], ((0, 0), (0, 98)))
    as1p = jnp.pad(params["enc_asrc1"], (0, 98))
    ad1p = jnp.pad(params["enc_adst1"], (0, 98))
    b1p = jnp.pad(params["enc_b1"], (0, 98))
    hw1, es1, ed1 = _mm_es_ed(out0, w1p, as1p, ad1p, act="elu")
    out1 = _edge_phase_sc(hw1, es1, ed1, ssrc_p, sdst_p, starts, b1p)

    enc_rep = _zero_rows_tc(out1, mask_nodes)

    w2p = jnp.pad(params["dec_W0"], ((0, 98), (0, 0)))

    hw2, es2, ed2 = _mm_es_ed(enc_rep, w2p, params["dec_asrc0"],
                              params["dec_adst0"], act=None)
    out2 = _edge_phase_sc(hw2, es2, ed2, ssrc_p, sdst_p, starts, params["dec_b0"])

    hw3, es3, ed3 = _mm_es_ed(out2, params["dec_W1"], params["dec_asrc1"],
                              params["dec_adst1"], act="elu")
    recon = _edge_phase_sc(hw3, es3, ed3, ssrc_p, sdst_p, starts, params["dec_b1"])

    loss = _sce_loss_tc(x, recon, mask_nodes)
    return loss, recon
